# baseline (speedup 1.0000x reference)
import functools
import math

import jax
import jax.numpy as jnp
from jax import lax
from jax.experimental import pallas as pl
from jax.experimental.pallas import tpu as pltpu
from jax.experimental.pallas import tpu_sc as plsc

N = 10000
E = 320000
D = 128
H = 8
DH = 16
DFF = 512

NC = 2
NS = 16
NW = NC * NS
EW = E // NW
CH = 40
NCHUNK = EW // CH
ROWW = D + 16
NPAD = 10240
RPT = NPAD // NS


def _qkv_body(feat_ref, wq_ref, wkv_ref, bq_ref, bkv_ref, q_ref, kv_ref):
    x = feat_ref[...]
    q_ref[...] = (jnp.dot(x, wq_ref[...], preferred_element_type=jnp.float32)
                  + bq_ref[...]) * (1.0 / math.sqrt(D))
    kv_ref[...] = jnp.dot(x, wkv_ref[...], preferred_element_type=jnp.float32) + bkv_ref[...]


def _qkv(feat, wq, wkv, bq, bkv):
    blk = 2000
    grid = N // blk
    return pl.pallas_call(
        _qkv_body,
        grid=(grid,),
        in_specs=[
            pl.BlockSpec((blk, D), lambda i: (i, 0)),
            pl.BlockSpec((D, D), lambda i: (0, 0)),
            pl.BlockSpec((D, 2 * D), lambda i: (0, 0)),
            pl.BlockSpec((1, D), lambda i: (0, 0)),
            pl.BlockSpec((1, 2 * D), lambda i: (0, 0)),
        ],
        out_specs=[
            pl.BlockSpec((blk, D), lambda i: (i, 0)),
            pl.BlockSpec((blk, 2 * D), lambda i: (i, 0)),
        ],
        out_shape=[
            jax.ShapeDtypeStruct((N, D), jnp.float32),
            jax.ShapeDtypeStruct((N, 2 * D), jnp.float32),
        ],
    )(feat, wq, wkv, bq, bkv)


def _edge_body(kv_hbm, q_hbm, sd_hbm, part_hbm,
               acc, sd_idx, kv_rows, q_rows, m_st,
               sem_i0, sem_i1, sem_kv0, sem_kv1, sem_q0, sem_q1):
    cid = lax.axis_index("c")
    sid = lax.axis_index("s")
    sem_i = (sem_i0, sem_i1)
    sem_kv = (sem_kv0, sem_kv1)
    sem_q = (sem_q0, sem_q1)

    def _zrow(i, _):
        for jj in range(ROWW // 16):
            m_st[i, pl.ds(jj * 16, 16)] = jnp.zeros((16,), jnp.float32)
        return _
    lax.fori_loop(0, CH, _zrow, None)

    def _zcp(b, _):
        pltpu.sync_copy(m_st, acc.at[pl.ds(sid * RPT + b * CH, CH)])
        return _
    lax.fori_loop(0, RPT // CH, _zcp, None)
    plsc.subcore_barrier()

    wid = sid * NC + cid
    lane = lax.iota(jnp.int32, 16)
    _gdn = lax.GatherDimensionNumbers(
        offset_dims=(), collapsed_slice_dims=(0,), start_index_map=(0,))

    def _shuf(x, idx):
        return lax.gather(x, idx, _gdn, (1,),
                          mode=lax.GatherScatterMode.PROMISE_IN_BOUNDS)

    cbase = wid * NCHUNK

    ix = [(lane ^ sh)[:, None] for sh in (8, 4, 2, 1)]
    b3z = (lane & 8) == 0
    b2z = (lane & 4) == 0
    b1z = (lane & 2) == 0
    _b = lane & 7
    idx_den = (2 * ((_b >> 2) & 1) + 4 * ((_b >> 1) & 1) + 8 * (_b & 1))[:, None]
    loh = [2 * ((h >> 2) & 1) + 4 * ((h >> 1) & 1) + 8 * (h & 1) for h in range(H)]
    idx_bc = [jnp.broadcast_to(jnp.int32(loh[h]), (16,))[:, None] for h in range(H)]

    def _start_idx(c, b):
        pltpu.async_copy(sd_hbm.at[pl.ds((cbase + c) * 2 * CH, 2 * CH)],
                         sd_idx.at[b], sem_i[b])

    def _wait_idx(b):
        pltpu.make_async_copy(sd_hbm.at[pl.ds(0, 2 * CH)],
                              sd_idx.at[b], sem_i[b]).wait()

    def _start_gathers(b):
        pltpu.async_copy(kv_hbm.at[sd_idx.at[b, pl.ds(0, CH)]],
                         kv_rows.at[b], sem_kv[b])
        pltpu.async_copy(q_hbm.at[sd_idx.at[b, pl.ds(CH, CH)]],
                         q_rows.at[b], sem_q[b])

    def _wait_gathers(b):
        pltpu.make_async_copy(kv_hbm.at[sd_idx.at[b, pl.ds(0, CH)]],
                              kv_rows.at[b], sem_kv[b]).wait()
        pltpu.make_async_copy(q_hbm.at[sd_idx.at[b, pl.ds(CH, CH)]],
                              q_rows.at[b], sem_q[b]).wait()

    def _stage(c, b):
        bn = 1 - b

        @pl.when(c + 1 < NCHUNK)
        def _p1():
            _wait_idx(bn)
            _start_gathers(bn)

        _wait_gathers(b)

        @plsc.parallel_loop(0, CH, unroll=4)
        def _edge(e):
            p = [kv_rows[b, e, pl.ds(h * DH, 16)] * q_rows[b, e, pl.ds(h * DH, 16)]
                 for h in range(H)]
            a = [x + _shuf(x, ix[0]) for x in p]
            m1 = [jnp.where(b3z, a[2 * i], a[2 * i + 1]) for i in range(4)]
            f4 = [x + _shuf(x, ix[1]) for x in m1]
            m2 = [jnp.where(b2z, f4[2 * j], f4[2 * j + 1]) for j in range(2)]
            g2 = [x + _shuf(x, ix[2]) for x in m2]
            m3 = jnp.where(b1z, g2[0], g2[1])
            ee = jnp.exp(m3 + _shuf(m3, ix[3]))
            m_st[e, pl.ds(D, 16)] = _shuf(ee, idx_den)
            for h in range(H):
                m_st[e, pl.ds(h * DH, 16)] = (
                    _shuf(ee, idx_bc[h]) * kv_rows[b, e, pl.ds(D + h * DH, 16)])

        pltpu.sync_copy(m_st, acc.at[sd_idx.at[b, pl.ds(CH, CH)]], add=True)

        @pl.when(c + 2 < NCHUNK)
        def _p2():
            _start_idx(c + 2, b)

    _start_idx(0, 0)
    _wait_idx(0)
    _start_gathers(0)
    _start_idx(1, 1)

    def _pair(t, _):
        _stage(2 * t, 0)
        _stage(2 * t + 1, 1)
        return _
    lax.fori_loop(0, NCHUNK // 2, _pair, None)
    plsc.subcore_barrier()

    pltpu.sync_copy(acc.at[pl.ds(sid * RPT, RPT)],
                    part_hbm.at[cid, pl.ds(sid * RPT, RPT)])


_edge_kernel = functools.partial(
    pl.kernel,
    out_type=jax.ShapeDtypeStruct((NC, NPAD, ROWW), jnp.float32),
    mesh=plsc.VectorSubcoreMesh(core_axis_name="c", subcore_axis_name="s"),
    compiler_params=pltpu.CompilerParams(use_tc_tiling_on_sc=False),
    scratch_types=[
        pltpu.VMEM_SHARED((NPAD, ROWW), jnp.float32),
        pltpu.VMEM((2, 2 * CH), jnp.int32),
        pltpu.VMEM((2, CH, 2 * D), jnp.float32),
        pltpu.VMEM((2, CH, D), jnp.float32),
        pltpu.VMEM((CH, ROWW), jnp.float32),
        pltpu.SemaphoreType.DMA,
        pltpu.SemaphoreType.DMA,
        pltpu.SemaphoreType.DMA,
        pltpu.SemaphoreType.DMA,
        pltpu.SemaphoreType.DMA,
        pltpu.SemaphoreType.DMA,
    ],
)(_edge_body)


def _fin_body(p_ref, feat_ref, lng_ref, lnb_ref, w1_ref, b1_ref, alpha_ref,
              w2_ref, b2_ref, out_ref):
    p = p_ref[0] + p_ref[1]
    ft2 = p[:, :D]
    den8 = p[:, D:D + H]
    hh = lax.broadcasted_iota(jnp.int32, (H, D), 0)
    jj = lax.broadcasted_iota(jnp.int32, (H, D), 1)
    sel = (jj // DH == hh).astype(jnp.float32)
    den = jnp.dot(den8, sel, preferred_element_type=jnp.float32)
    den = jnp.where(den > 0.0, den, 1.0)
    rst = ft2 / den + feat_ref[...]

    g = lng_ref[...]
    b = lnb_ref[...]

    def ln(x):
        mu = jnp.mean(x, axis=-1, keepdims=True)
        var = jnp.mean((x - mu) ** 2, axis=-1, keepdims=True)
        return (x - mu) / jnp.sqrt(var + 1e-5) * g + b

    rst = ln(rst)
    h1 = jnp.dot(rst, w1_ref[...], preferred_element_type=jnp.float32) + b1_ref[...]
    h1 = jnp.where(h1 >= 0.0, h1, alpha_ref[...] * h1)
    ffn = jnp.dot(h1, w2_ref[...], preferred_element_type=jnp.float32) + b2_ref[...]
    out_ref[...] = ln(rst + ffn)


def _final(part, feat, ln_g, ln_b, w1, b1, alpha, w2, b2):
    blk = 2000
    grid = N // blk
    return pl.pallas_call(
        _fin_body,
        grid=(grid,),
        in_specs=[
            pl.BlockSpec((NC, blk, ROWW), lambda i: (0, i, 0)),
            pl.BlockSpec((blk, D), lambda i: (i, 0)),
            pl.BlockSpec((1, D), lambda i: (0, 0)),
            pl.BlockSpec((1, D), lambda i: (0, 0)),
            pl.BlockSpec((D, DFF), lambda i: (0, 0)),
            pl.BlockSpec((1, DFF), lambda i: (0, 0)),
            pl.BlockSpec((1, DFF), lambda i: (0, 0)),
            pl.BlockSpec((DFF, D), lambda i: (0, 0)),
            pl.BlockSpec((1, D), lambda i: (0, 0)),
        ],
        out_specs=pl.BlockSpec((blk, D), lambda i: (i, 0)),
        out_shape=jax.ShapeDtypeStruct((N, D), jnp.float32),
    )(part, feat, ln_g, ln_b, w1, b1, alpha, w2, b2)


def kernel(feat, edge_index, Wq, bq, Wk, bk, Wv, bv, ln_g, ln_b, W1, bf1,
           alpha, W2, bf2):
    wkv = jnp.concatenate([Wk, Wv], axis=1)
    bkv = jnp.concatenate([bk, bv]).reshape(1, 2 * D)
    q, kv = _qkv(feat, Wq, wkv, bq.reshape(1, D), bkv)

    src = edge_index[0].astype(jnp.int32)
    dst = edge_index[1].astype(jnp.int32)
    sd = jnp.stack([src.reshape(-1, CH), dst.reshape(-1, CH)], axis=1).reshape(-1)
    part = _edge_kernel(kv, q, sd)

    return _final(part, feat, ln_g.reshape(1, D), ln_b.reshape(1, D),
                  W1, bf1.reshape(1, DFF), alpha.reshape(1, DFF),
                  W2, bf2.reshape(1, D))


# speedup vs baseline: 1.5452x; 1.5452x over previous
import functools
import math

import jax
import jax.numpy as jnp
from jax import lax
from jax.experimental import pallas as pl
from jax.experimental.pallas import tpu as pltpu
from jax.experimental.pallas import tpu_sc as plsc

N = 10000
E = 320000
D = 128
H = 8
DH = 16
DFF = 512

NC = 2
NS = 16
NW = NC * NS
EW = E // NW
CH = 40
NCHUNK = EW // CH
ROWW = D + 16
NPAD = 10240
RPT = NPAD // NS


def _qkv_body(feat_ref, wq_ref, wkv_ref, bq_ref, bkv_ref, q_ref, kv_ref):
    x = feat_ref[...]
    q_ref[...] = (jnp.dot(x, wq_ref[...], preferred_element_type=jnp.float32)
                  + bq_ref[...]) * (1.0 / math.sqrt(D))
    kv_ref[...] = jnp.dot(x, wkv_ref[...], preferred_element_type=jnp.float32) + bkv_ref[...]


def _qkv(feat, wq, wkv, bq, bkv):
    blk = 2000
    grid = N // blk
    return pl.pallas_call(
        _qkv_body,
        grid=(grid,),
        in_specs=[
            pl.BlockSpec((blk, D), lambda i: (i, 0)),
            pl.BlockSpec((D, D), lambda i: (0, 0)),
            pl.BlockSpec((D, 2 * D), lambda i: (0, 0)),
            pl.BlockSpec((1, D), lambda i: (0, 0)),
            pl.BlockSpec((1, 2 * D), lambda i: (0, 0)),
        ],
        out_specs=[
            pl.BlockSpec((blk, D), lambda i: (i, 0)),
            pl.BlockSpec((blk, 2 * D), lambda i: (i, 0)),
        ],
        out_shape=[
            jax.ShapeDtypeStruct((N, D), jnp.float32),
            jax.ShapeDtypeStruct((N, 2 * D), jnp.float32),
        ],
    )(feat, wq, wkv, bq, bkv)


def _edge_body(kv_hbm, q_hbm, src_hbm, dst_hbm, part_hbm,
               acc, sd_idx, kv_rows, q_rows, m_st,
               sem_i0, sem_i1, sem_kv0, sem_kv1, sem_q0, sem_q1):
    cid = lax.axis_index("c")
    sid = lax.axis_index("s")
    sem_i = (sem_i0, sem_i1)
    sem_kv = (sem_kv0, sem_kv1)
    sem_q = (sem_q0, sem_q1)

    def _zrow(i, _):
        for jj in range(ROWW // 16):
            m_st[i, pl.ds(jj * 16, 16)] = jnp.zeros((16,), jnp.float32)
        return _
    lax.fori_loop(0, CH, _zrow, None)

    def _zcp(b, _):
        pltpu.sync_copy(m_st, acc.at[pl.ds(sid * RPT + b * CH, CH)])
        return _
    lax.fori_loop(0, RPT // CH, _zcp, None)
    plsc.subcore_barrier()

    wid = sid * NC + cid
    lane = lax.iota(jnp.int32, 16)
    _gdn = lax.GatherDimensionNumbers(
        offset_dims=(), collapsed_slice_dims=(0,), start_index_map=(0,))

    def _shuf(x, idx):
        return lax.gather(x, idx, _gdn, (1,),
                          mode=lax.GatherScatterMode.PROMISE_IN_BOUNDS)

    cbase = wid * NCHUNK

    ix = [(lane ^ sh)[:, None] for sh in (8, 4, 2, 1)]
    b3z = (lane & 8) == 0
    b2z = (lane & 4) == 0
    b1z = (lane & 2) == 0
    _b = lane & 7
    idx_den = (2 * ((_b >> 2) & 1) + 4 * ((_b >> 1) & 1) + 8 * (_b & 1))[:, None]
    loh = [2 * ((h >> 2) & 1) + 4 * ((h >> 1) & 1) + 8 * (h & 1) for h in range(H)]
    idx_bc = [jnp.broadcast_to(jnp.int32(loh[h]), (16,))[:, None] for h in range(H)]

    def _start_idx(c, b):
        off = (cbase + c) * CH
        pltpu.async_copy(src_hbm.at[pl.ds(off, CH)],
                         sd_idx.at[b, pl.ds(0, CH)], sem_i[b])
        pltpu.async_copy(dst_hbm.at[pl.ds(off, CH)],
                         sd_idx.at[b, pl.ds(CH, CH)], sem_i[b])

    def _wait_idx(b):
        pltpu.make_async_copy(src_hbm.at[pl.ds(0, CH)],
                              sd_idx.at[b, pl.ds(0, CH)], sem_i[b]).wait()
        pltpu.make_async_copy(dst_hbm.at[pl.ds(0, CH)],
                              sd_idx.at[b, pl.ds(CH, CH)], sem_i[b]).wait()

    def _start_gathers(b):
        pltpu.async_copy(kv_hbm.at[sd_idx.at[b, pl.ds(0, CH)]],
                         kv_rows.at[b], sem_kv[b])
        pltpu.async_copy(q_hbm.at[sd_idx.at[b, pl.ds(CH, CH)]],
                         q_rows.at[b], sem_q[b])

    def _wait_gathers(b):
        pltpu.make_async_copy(kv_hbm.at[sd_idx.at[b, pl.ds(0, CH)]],
                              kv_rows.at[b], sem_kv[b]).wait()
        pltpu.make_async_copy(q_hbm.at[sd_idx.at[b, pl.ds(CH, CH)]],
                              q_rows.at[b], sem_q[b]).wait()

    def _stage(c, b):
        bn = 1 - b

        @pl.when(c + 1 < NCHUNK)
        def _p1():
            _wait_idx(bn)
            _start_gathers(bn)

        _wait_gathers(b)

        @plsc.parallel_loop(0, CH, unroll=2)
        def _edge(e):
            p = [kv_rows[b, e, pl.ds(h * DH, 16)] * q_rows[b, e, pl.ds(h * DH, 16)]
                 for h in range(H)]
            a = [x + _shuf(x, ix[0]) for x in p]
            m1 = [jnp.where(b3z, a[2 * i], a[2 * i + 1]) for i in range(4)]
            f4 = [x + _shuf(x, ix[1]) for x in m1]
            m2 = [jnp.where(b2z, f4[2 * j], f4[2 * j + 1]) for j in range(2)]
            g2 = [x + _shuf(x, ix[2]) for x in m2]
            m3 = jnp.where(b1z, g2[0], g2[1])
            ee = jnp.exp(m3 + _shuf(m3, ix[3]))
            m_st[e, pl.ds(D, 16)] = _shuf(ee, idx_den)
            for h in range(H):
                m_st[e, pl.ds(h * DH, 16)] = (
                    _shuf(ee, idx_bc[h]) * kv_rows[b, e, pl.ds(D + h * DH, 16)])

        pltpu.sync_copy(m_st, acc.at[sd_idx.at[b, pl.ds(CH, CH)]], add=True)

        @pl.when(c + 2 < NCHUNK)
        def _p2():
            _start_idx(c + 2, b)

    _start_idx(0, 0)
    _wait_idx(0)
    _start_gathers(0)
    _start_idx(1, 1)

    def _pair(t, _):
        _stage(2 * t, 0)
        _stage(2 * t + 1, 1)
        return _
    lax.fori_loop(0, NCHUNK // 2, _pair, None)
    plsc.subcore_barrier()

    pltpu.sync_copy(acc.at[pl.ds(sid * RPT, RPT)],
                    part_hbm.at[cid, pl.ds(sid * RPT, RPT)])


_edge_kernel = functools.partial(
    pl.kernel,
    out_type=jax.ShapeDtypeStruct((NC, NPAD, ROWW), jnp.float32),
    mesh=plsc.VectorSubcoreMesh(core_axis_name="c", subcore_axis_name="s"),
    compiler_params=pltpu.CompilerParams(use_tc_tiling_on_sc=False),
    scratch_types=[
        pltpu.VMEM_SHARED((NPAD, ROWW), jnp.float32),
        pltpu.VMEM((2, 2 * CH), jnp.int32),
        pltpu.VMEM((2, CH, 2 * D), jnp.float32),
        pltpu.VMEM((2, CH, D), jnp.float32),
        pltpu.VMEM((CH, ROWW), jnp.float32),
        pltpu.SemaphoreType.DMA,
        pltpu.SemaphoreType.DMA,
        pltpu.SemaphoreType.DMA,
        pltpu.SemaphoreType.DMA,
        pltpu.SemaphoreType.DMA,
        pltpu.SemaphoreType.DMA,
    ],
)(_edge_body)


def _fin_body(p_ref, feat_ref, lng_ref, lnb_ref, w1_ref, b1_ref, alpha_ref,
              w2_ref, b2_ref, out_ref):
    p = p_ref[0] + p_ref[1]
    ft2 = p[:, :D]
    den8 = p[:, D:D + H]
    hh = lax.broadcasted_iota(jnp.int32, (H, D), 0)
    jj = lax.broadcasted_iota(jnp.int32, (H, D), 1)
    sel = (jj // DH == hh).astype(jnp.float32)
    den = jnp.dot(den8, sel, preferred_element_type=jnp.float32)
    den = jnp.where(den > 0.0, den, 1.0)
    rst = ft2 / den + feat_ref[...]

    g = lng_ref[...]
    b = lnb_ref[...]

    def ln(x):
        mu = jnp.mean(x, axis=-1, keepdims=True)
        var = jnp.mean((x - mu) ** 2, axis=-1, keepdims=True)
        return (x - mu) / jnp.sqrt(var + 1e-5) * g + b

    rst = ln(rst)
    h1 = jnp.dot(rst, w1_ref[...], preferred_element_type=jnp.float32) + b1_ref[...]
    h1 = jnp.where(h1 >= 0.0, h1, alpha_ref[...] * h1)
    ffn = jnp.dot(h1, w2_ref[...], preferred_element_type=jnp.float32) + b2_ref[...]
    out_ref[...] = ln(rst + ffn)


def _final(part, feat, ln_g, ln_b, w1, b1, alpha, w2, b2):
    blk = 2000
    grid = N // blk
    return pl.pallas_call(
        _fin_body,
        grid=(grid,),
        in_specs=[
            pl.BlockSpec((NC, blk, ROWW), lambda i: (0, i, 0)),
            pl.BlockSpec((blk, D), lambda i: (i, 0)),
            pl.BlockSpec((1, D), lambda i: (0, 0)),
            pl.BlockSpec((1, D), lambda i: (0, 0)),
            pl.BlockSpec((D, DFF), lambda i: (0, 0)),
            pl.BlockSpec((1, DFF), lambda i: (0, 0)),
            pl.BlockSpec((1, DFF), lambda i: (0, 0)),
            pl.BlockSpec((DFF, D), lambda i: (0, 0)),
            pl.BlockSpec((1, D), lambda i: (0, 0)),
        ],
        out_specs=pl.BlockSpec((blk, D), lambda i: (i, 0)),
        out_shape=jax.ShapeDtypeStruct((N, D), jnp.float32),
    )(part, feat, ln_g, ln_b, w1, b1, alpha, w2, b2)


def kernel(feat, edge_index, Wq, bq, Wk, bk, Wv, bv, ln_g, ln_b, W1, bf1,
           alpha, W2, bf2):
    wkv = jnp.concatenate([Wk, Wv], axis=1)
    bkv = jnp.concatenate([bk, bv]).reshape(1, 2 * D)
    q, kv = _qkv(feat, Wq, wkv, bq.reshape(1, D), bkv)

    src = edge_index[0].astype(jnp.int32)
    dst = edge_index[1].astype(jnp.int32)
    part = _edge_kernel(kv, q, src, dst)

    return _final(part, feat, ln_g.reshape(1, D), ln_b.reshape(1, D),
                  W1, bf1.reshape(1, DFF), alpha.reshape(1, DFF),
                  W2, bf2.reshape(1, D))
